# baseline (speedup 1.0000x reference)
import jax
import jax.numpy as jnp
from jax.experimental import pallas as pl
from jax.experimental.pallas import tpu as pltpu

N = 16384
K = 512
BC = 1024
NBUF = 4
NCH = N // BC


def _body(t_hbm, out_hbm, buf, obuf, isems, osems):
    def _start(c, b):
        @pl.when(c < NCH)
        def _():
            pltpu.make_async_copy(
                t_hbm.at[:, pl.ds(c * BC, BC), :], buf.at[b], isems.at[b]
            ).start()

    def _wait(c, b):
        pltpu.make_async_copy(
            t_hbm.at[:, pl.ds(c * BC, BC), :], buf.at[b], isems.at[b]
        ).wait()

    for b in range(NBUF - 1):
        _start(b, b)

    def _chunk(ci):
        for b in range(NBUF):
            cur = ci + b
            _start(cur + NBUF - 1, (b + NBUF - 1) % NBUF)
            _wait(cur, b)
            d = buf[b, 0] + buf[b, 1] - buf[b, 2]
            @pl.when(cur >= NBUF)
            def _():
                pltpu.make_async_copy(
                    obuf.at[b], out_hbm.at[pl.ds((cur - NBUF) * BC, BC)],
                    osems.at[b],
                ).wait()
            obuf[b] = -jnp.sqrt(jnp.sum(d * d, axis=1))
            pltpu.make_async_copy(
                obuf.at[b], out_hbm.at[pl.ds(cur * BC, BC)], osems.at[b]
            ).start()

    pl.loop(0, NCH, step=NBUF)(_chunk)

    for b in range(NBUF):
        cur = NCH - NBUF + b
        pltpu.make_async_copy(
            obuf.at[b], out_hbm.at[pl.ds(cur * BC, BC)], osems.at[b]
        ).wait()


def kernel(triples):
    return pl.pallas_call(
        _body,
        in_specs=[pl.BlockSpec(memory_space=pltpu.HBM)],
        out_specs=pl.BlockSpec(memory_space=pltpu.HBM),
        out_shape=jax.ShapeDtypeStruct((N,), jnp.float32),
        scratch_shapes=[
            pltpu.VMEM((NBUF, 3, BC, K), jnp.float32),
            pltpu.VMEM((NBUF, BC), jnp.float32),
            pltpu.SemaphoreType.DMA((NBUF,)),
            pltpu.SemaphoreType.DMA((NBUF,)),
        ],
    )(triples)


# speedup vs baseline: 1.0139x; 1.0139x over previous
import jax
import jax.numpy as jnp
from jax.experimental import pallas as pl
from jax.experimental.pallas import tpu as pltpu

N = 16384
K = 512
BC = 1024
NBUF = 4
NCH = N // BC


def _body(t_hbm, out_hbm, buf, obuf, isems, osems):
    def _start(c, b):
        @pl.when(c < NCH)
        def _():
            for pln in range(3):
                pltpu.make_async_copy(
                    t_hbm.at[pln, pl.ds(c * BC, BC), :], buf.at[b, pln],
                    isems.at[b],
                ).start()

    def _wait(c, b):
        for pln in range(3):
            pltpu.make_async_copy(
                t_hbm.at[pln, pl.ds(c * BC, BC), :], buf.at[b, pln],
                isems.at[b],
            ).wait()

    for b in range(NBUF - 1):
        _start(b, b)

    def _chunk(ci):
        for b in range(NBUF):
            cur = ci + b
            _start(cur + NBUF - 1, (b + NBUF - 1) % NBUF)
            _wait(cur, b)
            d = buf[b, 0] + buf[b, 1] - buf[b, 2]
            @pl.when(cur >= NBUF)
            def _():
                pltpu.make_async_copy(
                    obuf.at[b], out_hbm.at[pl.ds((cur - NBUF) * BC, BC)],
                    osems.at[b],
                ).wait()
            obuf[b] = -jnp.sqrt(jnp.sum(d * d, axis=1))
            pltpu.make_async_copy(
                obuf.at[b], out_hbm.at[pl.ds(cur * BC, BC)], osems.at[b]
            ).start()

    pl.loop(0, NCH, step=NBUF)(_chunk)

    for b in range(NBUF):
        cur = NCH - NBUF + b
        pltpu.make_async_copy(
            obuf.at[b], out_hbm.at[pl.ds(cur * BC, BC)], osems.at[b]
        ).wait()


def kernel(triples):
    return pl.pallas_call(
        _body,
        in_specs=[pl.BlockSpec(memory_space=pltpu.HBM)],
        out_specs=pl.BlockSpec(memory_space=pltpu.HBM),
        out_shape=jax.ShapeDtypeStruct((N,), jnp.float32),
        scratch_shapes=[
            pltpu.VMEM((NBUF, 3, BC, K), jnp.float32),
            pltpu.VMEM((NBUF, BC), jnp.float32),
            pltpu.SemaphoreType.DMA((NBUF,)),
            pltpu.SemaphoreType.DMA((NBUF,)),
        ],
    )(triples)
